# Initial kernel scaffold; baseline (speedup 1.0000x reference)
import jax
import jax.numpy as jnp
from jax.experimental import pallas as pl


def kernel(new_node, rbf, edge_f, edge_index, W1, b1, W2, b2):
    raise NotImplementedError("write your pallas kernel here")


# speedup vs baseline: 1.4741x; 1.4741x over previous
import functools

import jax
import jax.numpy as jnp
from jax import lax
from jax.experimental import pallas as pl
from jax.experimental.pallas import tpu as pltpu
from jax.experimental.pallas import tpu_sc as plsc

N_NODES = 50000
N_EDGES = 800000
RBF_DIM = 128
DIM = 64
BETA = 0.5
THRESHOLD = 14.0


MLP_BLK = 2000


def _mlp_body(rbf_ref, w1_ref, b1_ref, w2_ref, b2_ref, h_ref):
    x = rbf_ref[...]
    h = jnp.dot(x, w1_ref[...], preferred_element_type=jnp.float32) + b1_ref[...]
    bx = BETA * h
    sp = (jnp.maximum(bx, 0.0) + jnp.log1p(jnp.exp(-jnp.abs(bx)))) / BETA
    h = jnp.where(bx > THRESHOLD, h, sp)
    h = jnp.dot(h, w2_ref[...], preferred_element_type=jnp.float32) + b2_ref[...]
    h_ref[...] = h


def _mlp(rbf, W1, b1, W2, b2):
    n = rbf.shape[0]
    grid = n // MLP_BLK
    return pl.pallas_call(
        _mlp_body,
        grid=(grid,),
        in_specs=[
            pl.BlockSpec((MLP_BLK, RBF_DIM), lambda i: (i, 0)),
            pl.BlockSpec((RBF_DIM, DIM), lambda i: (0, 0)),
            pl.BlockSpec((DIM,), lambda i: (0,)),
            pl.BlockSpec((DIM, DIM), lambda i: (0, 0)),
            pl.BlockSpec((DIM,), lambda i: (0,)),
        ],
        out_specs=pl.BlockSpec((MLP_BLK, DIM), lambda i: (i, 0)),
        out_shape=jax.ShapeDtypeStruct((n, DIM), jnp.float32),
    )(rbf, W1, b1, W2, b2)


NC = 2
NS = 16
CHUNK = 80
EDGES_PER_TILE = N_EDGES // NS
N_CHUNKS = EDGES_PER_TILE // CHUNK
HALF = N_NODES // NC
ACC_ROWS = 25088
ZROWS = ACC_ROWS // NS
OUT_ROWS = 1560


def _sc_body(nn_hbm, h_hbm, ef_hbm, src_hbm, dst_hbm, zero_hbm, out_hbm,
             src_v, dst_v, idx_v, nn_v, h_v, ef_v, acc_sh,
             sem_g, sem_h, sem_e):
    c = lax.axis_index("c")
    s = lax.axis_index("s")
    base_node = c * HALF

    pltpu.sync_copy(zero_hbm, acc_sh.at[pl.ds(s * ZROWS, ZROWS)])
    plsc.subcore_barrier()

    def chunk_body(j, _):
        e0 = s * EDGES_PER_TILE + j * CHUNK
        pltpu.sync_copy(src_hbm.at[pl.ds(e0, CHUNK)], src_v)
        pltpu.sync_copy(dst_hbm.at[pl.ds(e0, CHUNK)], dst_v)
        cp_g = pltpu.async_copy(nn_hbm.at[src_v], nn_v, sem_g)
        cp_h = pltpu.async_copy(h_hbm.at[pl.ds(e0, CHUNK)], h_v, sem_h)
        cp_e = pltpu.async_copy(ef_hbm.at[pl.ds(e0, CHUNK)], ef_v, sem_e)
        for i in range(CHUNK // 16):
            d = dst_v[pl.ds(i * 16, 16)]
            ld = d - base_node
            own = (ld >= 0) & (ld < HALF)
            garb = HALF + jnp.bitwise_and(d, 63)
            idx_v[pl.ds(i * 16, 16)] = jnp.where(own, ld, garb)
        cp_g.wait()
        cp_h.wait()
        cp_e.wait()

        def row_body(r, _):
            for jc in range(DIM // 16):
                sl = pl.ds(jc * 16, 16)
                ef_v[r, sl] = nn_v[r, sl] * h_v[r, sl] + ef_v[r, sl]
            return ()

        lax.fori_loop(0, CHUNK, row_body, (), unroll=4)
        pltpu.sync_copy(ef_v, acc_sh.at[idx_v], add=True)
        return ()

    lax.fori_loop(0, N_CHUNKS, chunk_body, ())
    plsc.subcore_barrier()

    pltpu.sync_copy(acc_sh.at[pl.ds(s * OUT_ROWS, OUT_ROWS)],
                    out_hbm.at[pl.ds(base_node + s * OUT_ROWS, OUT_ROWS)])

    @pl.when(s == 0)
    def _tail():
        pltpu.sync_copy(acc_sh.at[pl.ds(NS * OUT_ROWS, HALF - NS * OUT_ROWS)],
                        out_hbm.at[pl.ds(base_node + NS * OUT_ROWS,
                                         HALF - NS * OUT_ROWS)])


def _sc_scatter(new_node, h, edge_f, src, dst, zeros):
    mesh = plsc.VectorSubcoreMesh(core_axis_name="c", subcore_axis_name="s")
    kfn = pl.kernel(
        _sc_body,
        out_type=jax.ShapeDtypeStruct((N_NODES, DIM), jnp.float32),
        mesh=mesh,
        compiler_params=pltpu.CompilerParams(use_tc_tiling_on_sc=False),
        scratch_types=[
            pltpu.VMEM((CHUNK,), jnp.int32),
            pltpu.VMEM((CHUNK,), jnp.int32),
            pltpu.VMEM((CHUNK,), jnp.int32),
            pltpu.VMEM((CHUNK, DIM), jnp.float32),
            pltpu.VMEM((CHUNK, DIM), jnp.float32),
            pltpu.VMEM((CHUNK, DIM), jnp.float32),
            pltpu.VMEM_SHARED((ACC_ROWS, DIM), jnp.float32),
            pltpu.SemaphoreType.DMA,
            pltpu.SemaphoreType.DMA,
            pltpu.SemaphoreType.DMA,
        ],
    )
    return kfn(new_node, h, edge_f, src, dst, zeros)


def kernel(new_node, rbf, edge_f, edge_index, W1, b1, W2, b2):
    src = edge_index[0].astype(jnp.int32)
    dst = edge_index[1].astype(jnp.int32)
    h = _mlp(rbf, W1, b1, W2, b2)
    zeros = jnp.zeros((ZROWS, DIM), jnp.float32)
    return _sc_scatter(new_node, h, edge_f, src, dst, zeros)
